# Initial kernel scaffold; baseline (speedup 1.0000x reference)
import jax
import jax.numpy as jnp
from jax.experimental import pallas as pl


def kernel(rois):
    raise NotImplementedError("write your pallas kernel here")


# speedup vs baseline: 10.0938x; 10.0938x over previous
import jax
import jax.numpy as jnp
from jax.experimental import pallas as pl

_F = 250
_NB = 20
_N = _F * _NB
_T = 640
_G = (_N + _T - 1) // _T


def _tile_kernel(a_ref, bt_ref, o_ref):
    i = pl.program_id(0)
    j = pl.program_id(1)

    @pl.when(i != j)
    def _off_diag():
        o_ref[...] = jnp.zeros_like(o_ref)

    @pl.when(i == j)
    def _diag():
        a = a_ref[...]
        bt = bt_ref[...]
        ax1 = a[:, 0:1]
        ay1 = a[:, 1:2]
        ax2 = a[:, 2:3]
        ay2 = a[:, 3:4]
        bx1 = bt[0:1, :]
        by1 = bt[1:2, :]
        bx2 = bt[2:3, :]

        inter_x1 = jnp.maximum(ax1, bx1)
        inter_x2 = jnp.minimum(ax2, bx2)
        inter_y1 = jnp.maximum(ay1, by1)
        inter_y2 = jnp.minimum(ay2, bt[3:4, :])
        inter_area = (
            jnp.maximum(inter_x2 - inter_x1, 0.0)
            * jnp.maximum(inter_y2 - inter_y1, 0.0)
        )
        boxa_area = (ax2 - ax1 + 1.0) * (ay2 - ay1 + 1.0)
        boxb_area = (bx2 - bx1 + 1.0) * (bx2 - by1 + 1.0)
        iou = inter_area / (boxa_area + boxb_area - inter_area)

        r = jax.lax.broadcasted_iota(jnp.int32, (_T, _T), 0) // _NB
        c = jax.lax.broadcasted_iota(jnp.int32, (_T, _T), 1) // _NB
        gb = (_T // _NB) * i + r
        mask = (r == c) & (gb != 248)
        o_ref[...] = jnp.where(mask, iou, 0.0)


def kernel(rois):
    a_rows = jnp.roll(rois, -1, axis=0).reshape(_N, 4)
    b_cols = jnp.roll(rois, -2, axis=0).reshape(_N, 4).T

    out = pl.pallas_call(
        _tile_kernel,
        grid=(_G, _G),
        in_specs=[
            pl.BlockSpec((_T, 4), lambda i, j: (i, 0)),
            pl.BlockSpec((4, _T), lambda i, j: (0, j)),
        ],
        out_specs=pl.BlockSpec((_T, _T), lambda i, j: (i, j)),
        out_shape=jax.ShapeDtypeStruct((_N, _N), jnp.float32),
    )(a_rows, b_cols)
    return out.reshape(1, _N, _N)


# speedup vs baseline: 16.3284x; 1.6177x over previous
import jax
import jax.numpy as jnp
from jax.experimental import pallas as pl

_F = 250
_NB = 20
_N = _F * _NB
_T = 640
_G = (_N + _T - 1) // _T


def _strip_kernel(a_ref, bt_ref, o_ref):
    s = pl.program_id(0)

    a = a_ref[...]
    bt = bt_ref[...]
    ax1 = a[:, 0:1]
    ay1 = a[:, 1:2]
    ax2 = a[:, 2:3]
    ay2 = a[:, 3:4]
    bx1 = bt[0:1, :]
    by1 = bt[1:2, :]
    bx2 = bt[2:3, :]
    by2 = bt[3:4, :]

    inter_x1 = jnp.maximum(ax1, bx1)
    inter_x2 = jnp.minimum(ax2, bx2)
    inter_y1 = jnp.maximum(ay1, by1)
    inter_y2 = jnp.minimum(ay2, by2)
    inter_area = (
        jnp.maximum(inter_x2 - inter_x1, 0.0)
        * jnp.maximum(inter_y2 - inter_y1, 0.0)
    )
    boxa_area = (ax2 - ax1 + 1.0) * (ay2 - ay1 + 1.0)
    boxb_area = (bx2 - bx1 + 1.0) * (bx2 - by1 + 1.0)
    iou = inter_area / (boxa_area + boxb_area - inter_area)

    r = jax.lax.broadcasted_iota(jnp.int32, (_T, _T), 0) // _NB
    c = jax.lax.broadcasted_iota(jnp.int32, (_T, _T), 1) // _NB
    gb = (_T // _NB) * s + r
    mask = (r == c) & (gb != 248)
    tile = jnp.where(mask, iou, 0.0)

    o_ref[...] = jnp.zeros_like(o_ref)

    @pl.when(s < _G - 1)
    def _full():
        o_ref[:, pl.ds(s * _T, _T)] = tile

    @pl.when(s == _G - 1)
    def _last():
        o_ref[:, pl.ds(s * _T, _N - (_G - 1) * _T)] = tile[:, : _N - (_G - 1) * _T]


def kernel(rois):
    a_rows = jnp.roll(rois, -1, axis=0).reshape(_N, 4)
    b_cols = jnp.roll(rois, -2, axis=0).reshape(_N, 4).T

    out = pl.pallas_call(
        _strip_kernel,
        grid=(_G,),
        in_specs=[
            pl.BlockSpec((_T, 4), lambda s: (s, 0)),
            pl.BlockSpec((4, _T), lambda s: (0, s)),
        ],
        out_specs=pl.BlockSpec((_T, _N), lambda s: (s, 0)),
        out_shape=jax.ShapeDtypeStruct((_N, _N), jnp.float32),
    )(a_rows, b_cols)
    return out.reshape(1, _N, _N)
